# baseline (speedup 1.0000x reference)
import functools

import jax
import jax.numpy as jnp
from jax import lax
from jax.experimental import pallas as pl
from jax.experimental.pallas import tpu as pltpu
from jax.experimental.pallas import tpu_sc as plsc

EPS = 1e-5
D = 128
CH = 128


def _proj_body(x_ref, wa_ref, wb_ref, a_ref, b_ref):
    x = x_ref[...]
    a_ref[...] = jnp.dot(x, wa_ref[...], preferred_element_type=jnp.float32)
    b_ref[...] = jnp.dot(x, wb_ref[...], preferred_element_type=jnp.float32)


def _node_projections(x, wa, wb):
    n = x.shape[0]
    bn = 2000 if n % 2000 == 0 else n
    grid = n // bn
    return pl.pallas_call(
        _proj_body,
        grid=(grid,),
        in_specs=[
            pl.BlockSpec((bn, D), lambda i: (i, 0)),
            pl.BlockSpec((D, D), lambda i: (0, 0)),
            pl.BlockSpec((D, D), lambda i: (0, 0)),
        ],
        out_specs=[
            pl.BlockSpec((bn, D), lambda i: (i, 0)),
            pl.BlockSpec((bn, D), lambda i: (i, 0)),
        ],
        out_shape=[
            jax.ShapeDtypeStruct((n, D), jnp.float32),
            jax.ShapeDtypeStruct((n, D), jnp.float32),
        ],
    )(x, wa, wb)


def _sc_gather(a, b, src, dst):
    e = src.shape[0]
    info = plsc.get_sparse_core_info()
    nc, ns = info.num_cores, info.num_subcores
    nw = nc * ns
    iters = (e + nw * CH - 1) // (nw * CH)
    ep = iters * nw * CH

    pad = ep - e
    srcp = jnp.concatenate([src, jnp.zeros((pad,), jnp.int32)])
    dstp = jnp.concatenate([dst, jnp.zeros((pad,), jnp.int32)])
    ei = (jnp.stack([srcp, dstp])
          .reshape(2, iters, nw, CH)
          .transpose(2, 1, 0, 3))

    mesh = plsc.VectorSubcoreMesh(core_axis_name="c", subcore_axis_name="s")

    @functools.partial(
        pl.kernel,
        mesh=mesh,
        out_type=jax.ShapeDtypeStruct((ep, D), jnp.float32),
        scratch_types=[
            pltpu.VMEM((2, 2, CH), jnp.int32),
            pltpu.VMEM((2, CH, D), jnp.float32),
            pltpu.VMEM((2, CH, D), jnp.float32),
        ] + [pltpu.SemaphoreType.DMA] * 8,
    )
    def gather_kernel(a_hbm, b_hbm, ei_hbm, g_hbm,
                      idxr, bufa, bufb,
                      sga0, sga1, sgb0, sgb1, ssa0, ssa1, si0, si1):
        sga = (sga0, sga1)
        sgb = (sgb0, sgb1)
        ssa = (ssa0, ssa1)
        si = (si0, si1)
        wid = lax.axis_index("s") * nc + lax.axis_index("c")

        def stage_idx(i, slot):
            pltpu.async_copy(ei_hbm.at[wid, i], idxr.at[slot], si[slot])

        def wait_idx(slot):
            pltpu.make_async_copy(ei_hbm.at[0, 0], idxr.at[slot], si[slot]).wait()

        def gather(i, slot):
            pltpu.async_copy(a_hbm.at[idxr.at[slot, 0]], bufa.at[slot], sga[slot])
            pltpu.async_copy(b_hbm.at[idxr.at[slot, 1]], bufb.at[slot], sgb[slot])

        def wait_gathers(slot):
            pltpu.make_async_copy(a_hbm.at[pl.ds(0, CH)], bufa.at[slot], sga[slot]).wait()
            pltpu.make_async_copy(b_hbm.at[pl.ds(0, CH)], bufb.at[slot], sgb[slot]).wait()

        def add_rows(slot):
            def rows(r2, carry):
                for r8 in range(2):
                    for c in range(D // 16):
                        r = r2 * 2 + r8
                        sl = pl.ds(c * 16, 16)
                        bufa[slot, r, sl] = bufa[slot, r, sl] + bufb[slot, r, sl]
                return carry

            lax.fori_loop(0, CH // 2, rows, 0)

        def scatter(i, slot):
            base = (wid + i * nw) * CH
            pltpu.async_copy(bufa.at[slot], g_hbm.at[pl.ds(base, CH)], ssa[slot])

        def wait_scatters(slot):
            pltpu.make_async_copy(bufa.at[slot], g_hbm.at[pl.ds(0, CH)], ssa[slot]).wait()

        def step(i, slot, first):
            nslot = 1 - slot
            wait_gathers(slot)

            @pl.when(i + 1 < iters)
            def _():
                wait_idx(nslot)
                if not first:
                    wait_scatters(nslot)
                gather(i + 1, nslot)

            @pl.when(i + 2 < iters)
            def _():
                stage_idx(i + 2, slot)

            add_rows(slot)
            scatter(i, slot)

        pltpu.sync_copy(ei_hbm.at[wid, 0], idxr.at[0])
        stage_idx(1, 1)
        gather(0, 0)
        step(0, 0, first=True)

        def loop_body(j, carry):
            step(2 * j + 1, 1, first=False)
            step(2 * j + 2, 0, first=False)
            return carry

        lax.fori_loop(0, iters // 2, loop_body, 0)

        wait_scatters(0)
        wait_scatters(1)

    return gather_kernel(a, b, ei)


def _mlp_body(g_ref, e_ref, wc_ref, b1_ref, g1_ref, be1_ref,
              w2_ref, b2_ref, o_ref):
    eb = e_ref[...]
    h = (g_ref[...]
         + jnp.dot(eb, wc_ref[...], preferred_element_type=jnp.float32)
         + b1_ref[...])
    m = jnp.mean(h, axis=-1, keepdims=True)
    v = jnp.mean((h - m) ** 2, axis=-1, keepdims=True)
    hn = (h - m) / jnp.sqrt(v + EPS) * g1_ref[...] + be1_ref[...]
    hr = jnp.maximum(hn, 0.0)
    o_ref[...] = (eb + jnp.dot(hr, w2_ref[...], preferred_element_type=jnp.float32)
                  + b2_ref[...])


def _edge_mlp(g, edge, wc, b1, g1, be1, w2, b2):
    e = edge.shape[0]
    be = 2000 if e % 2000 == 0 else e
    grid = e // be
    row = lambda v: v.reshape(1, D)
    vec_spec = pl.BlockSpec((1, D), lambda i: (0, 0))
    mat_spec = pl.BlockSpec((D, D), lambda i: (0, 0))
    blk_spec = pl.BlockSpec((be, D), lambda i: (i, 0))
    return pl.pallas_call(
        _mlp_body,
        grid=(grid,),
        in_specs=[blk_spec, blk_spec, mat_spec,
                  vec_spec, vec_spec, vec_spec, mat_spec, vec_spec],
        out_specs=blk_spec,
        out_shape=jax.ShapeDtypeStruct((e, D), jnp.float32),
    )(g, edge, wc, row(b1), row(g1), row(be1), w2, row(b2))


def kernel(x, edge_index, edge, W1, b1, g1, be1, W2, b2):
    src = edge_index[0]
    dst = edge_index[1]
    wa, wb, wc = W1[:D], W1[D:2 * D], W1[2 * D:]
    a, b = _node_projections(x, wa, wb)
    g = _sc_gather(a, b, src, dst)
    return _edge_mlp(g, edge, wc, b1, g1, be1, W2, b2)


# speedup vs baseline: 1.4302x; 1.4302x over previous
import functools

import jax
import jax.numpy as jnp
from jax import lax
from jax.experimental import pallas as pl
from jax.experimental.pallas import tpu as pltpu
from jax.experimental.pallas import tpu_sc as plsc

EPS = 1e-5
D = 128
CH = 128
CHUNKS = 4
BE = 2000


def _proj_body(x_ref, wa_ref, wb_ref, a_ref, b_ref):
    x = x_ref[...]
    a_ref[...] = jnp.dot(x, wa_ref[...], preferred_element_type=jnp.float32)
    b_ref[...] = jnp.dot(x, wb_ref[...], preferred_element_type=jnp.float32)


def _node_projections(x, wa, wb):
    n = x.shape[0]
    bn = 2000 if n % 2000 == 0 else n
    grid = n // bn
    return pl.pallas_call(
        _proj_body,
        grid=(grid,),
        in_specs=[
            pl.BlockSpec((bn, D), lambda i: (i, 0)),
            pl.BlockSpec((D, D), lambda i: (0, 0)),
            pl.BlockSpec((D, D), lambda i: (0, 0)),
        ],
        out_specs=[
            pl.BlockSpec((bn, D), lambda i: (i, 0)),
            pl.BlockSpec((bn, D), lambda i: (i, 0)),
        ],
        out_shape=[
            jax.ShapeDtypeStruct((n, D), jnp.float32),
            jax.ShapeDtypeStruct((n, D), jnp.float32),
        ],
    )(x, wa, wb)


def _sc_gather(a, b, src, dst):
    e = src.shape[0]
    info = plsc.get_sparse_core_info()
    nc, ns = info.num_cores, info.num_subcores
    nw = nc * ns
    total_chunks = e // CH
    iters = (total_chunks + nw - 1) // nw
    mesh = plsc.VectorSubcoreMesh(core_axis_name="c", subcore_axis_name="s")

    @functools.partial(
        pl.kernel,
        mesh=mesh,
        out_type=jax.ShapeDtypeStruct((e, D), jnp.float32),
        scratch_types=[
            pltpu.VMEM((2, CH), jnp.int32),
            pltpu.VMEM((2, CH), jnp.int32),
            pltpu.VMEM((2, CH, D), jnp.float32),
            pltpu.VMEM((2, CH, D), jnp.float32),
        ] + [pltpu.SemaphoreType.DMA] * 6,
    )
    def gather_kernel(a_hbm, b_hbm, src_hbm, dst_hbm, g_hbm,
                      idxs, idxd, bufa, bufb,
                      sga0, sga1, sgb0, sgb1, ssa0, ssa1):
        sga = (sga0, sga1)
        sgb = (sgb0, sgb1)
        ssa = (ssa0, ssa1)
        wid = lax.axis_index("s") * nc + lax.axis_index("c")

        def valid(i):
            return wid + i * nw < total_chunks

        def stage_and_gather(i, slot):
            base = (wid + i * nw) * CH
            pltpu.sync_copy(src_hbm.at[pl.ds(base, CH)], idxs.at[slot])
            pltpu.sync_copy(dst_hbm.at[pl.ds(base, CH)], idxd.at[slot])
            pltpu.async_copy(a_hbm.at[idxs.at[slot]], bufa.at[slot], sga[slot])
            pltpu.async_copy(b_hbm.at[idxd.at[slot]], bufb.at[slot], sgb[slot])

        def wait_gathers(slot):
            pltpu.make_async_copy(a_hbm.at[pl.ds(0, CH)], bufa.at[slot], sga[slot]).wait()
            pltpu.make_async_copy(b_hbm.at[pl.ds(0, CH)], bufb.at[slot], sgb[slot]).wait()

        def add_rows(slot):
            def row(r, carry):
                for c in range(D // 16):
                    sl = pl.ds(c * 16, 16)
                    bufa[slot, r, sl] = bufa[slot, r, sl] + bufb[slot, r, sl]
                return carry

            lax.fori_loop(0, CH, row, 0)

        def scatter(i, slot):
            base = (wid + i * nw) * CH
            pltpu.async_copy(bufa.at[slot], g_hbm.at[pl.ds(base, CH)], ssa[slot])

        def wait_scatters(slot):
            pltpu.make_async_copy(bufa.at[slot], g_hbm.at[pl.ds(0, CH)], ssa[slot]).wait()

        def step(i, slot, first):
            nslot = 1 - slot

            @pl.when(valid(i))
            def _():
                wait_gathers(slot)

            @pl.when(valid(i + 1))
            def _():
                if not first:
                    wait_scatters(nslot)
                stage_and_gather(i + 1, nslot)

            @pl.when(valid(i))
            def _():
                add_rows(slot)
                scatter(i, slot)

        stage_and_gather(0, 0)
        step(0, 0, first=True)

        def loop_body(j, carry):
            step(2 * j + 1, 1, first=False)
            step(2 * j + 2, 0, first=False)
            return carry

        lax.fori_loop(0, (iters + 1) // 2, loop_body, 0)

        wait_scatters(0)
        wait_scatters(1)

    return gather_kernel(a, b, src, dst)


def _mlp_body(prev_ref, g_ref, e_ref, wc_ref, b1_ref, g1_ref, be1_ref,
              w2_ref, b2_ref, o_ref):
    del prev_ref
    eb = e_ref[...]
    h = (g_ref[...]
         + jnp.dot(eb, wc_ref[...], preferred_element_type=jnp.float32)
         + b1_ref[...])
    m = jnp.mean(h, axis=-1, keepdims=True)
    v = jnp.mean((h - m) ** 2, axis=-1, keepdims=True)
    hn = (h - m) / jnp.sqrt(v + EPS) * g1_ref[...] + be1_ref[...]
    hr = jnp.maximum(hn, 0.0)
    o_ref[...] = (eb + jnp.dot(hr, w2_ref[...], preferred_element_type=jnp.float32)
                  + b2_ref[...])


def _edge_mlp_chunk(out_prev, g, edge, wc, b1, g1, be1, w2, b2, blk0, nblk):
    e = edge.shape[0]
    row = lambda v: v.reshape(1, D)
    vec_spec = pl.BlockSpec((1, D), lambda i: (0, 0))
    mat_spec = pl.BlockSpec((D, D), lambda i: (0, 0))
    out_spec = pl.BlockSpec((BE, D), lambda i: (blk0 + i, 0))
    body = _mlp_body if out_prev is not None else functools.partial(_mlp_body, None)
    prev_spec = ([pl.BlockSpec(memory_space=pl.ANY)]
                 if out_prev is not None else [])
    prev_arg = (out_prev,) if out_prev is not None else ()
    return pl.pallas_call(
        body,
        grid=(nblk,),
        in_specs=prev_spec + [
            pl.BlockSpec((BE, D), lambda i: (i, 0)),
            out_spec, mat_spec,
            vec_spec, vec_spec, vec_spec, mat_spec, vec_spec],
        out_specs=out_spec,
        out_shape=jax.ShapeDtypeStruct((e, D), jnp.float32),
        input_output_aliases={0: 0} if out_prev is not None else {},
    )(*prev_arg, g, edge, wc, row(b1), row(g1), row(be1), w2, row(b2))


def kernel(x, edge_index, edge, W1, b1, g1, be1, W2, b2):
    e = edge.shape[0]
    src = edge_index[0]
    dst = edge_index[1]
    wa, wb, wc = W1[:D], W1[D:2 * D], W1[2 * D:]
    a, b = _node_projections(x, wa, wb)

    ec = e // CHUNKS
    assert ec % BE == 0 and ec % CH == 0
    nblk = ec // BE
    out = None
    for k in range(CHUNKS):
        sl = slice(k * ec, (k + 1) * ec)
        g = _sc_gather(a, b, src[sl], dst[sl])
        out = _edge_mlp_chunk(out, g, edge, wc, b1, g1, be1, W2, b2,
                              k * nblk, nblk)
    return out


# speedup vs baseline: 1.4623x; 1.0224x over previous
import functools

import jax
import jax.numpy as jnp
from jax import lax
from jax.experimental import pallas as pl
from jax.experimental.pallas import tpu as pltpu
from jax.experimental.pallas import tpu_sc as plsc

EPS = 1e-5
D = 128
CH = 128
CHUNK_SIZES = (32000, 96000, 96000, 64000, 32000)
BE = 4000


def _proj_body(x_ref, wa_ref, wb_ref, a_ref, b_ref):
    x = x_ref[...]
    a_ref[...] = jnp.dot(x, wa_ref[...], preferred_element_type=jnp.float32)
    b_ref[...] = jnp.dot(x, wb_ref[...], preferred_element_type=jnp.float32)


def _node_projections(x, wa, wb):
    n = x.shape[0]
    bn = 2000 if n % 2000 == 0 else n
    grid = n // bn
    return pl.pallas_call(
        _proj_body,
        grid=(grid,),
        in_specs=[
            pl.BlockSpec((bn, D), lambda i: (i, 0)),
            pl.BlockSpec((D, D), lambda i: (0, 0)),
            pl.BlockSpec((D, D), lambda i: (0, 0)),
        ],
        out_specs=[
            pl.BlockSpec((bn, D), lambda i: (i, 0)),
            pl.BlockSpec((bn, D), lambda i: (i, 0)),
        ],
        out_shape=[
            jax.ShapeDtypeStruct((n, D), jnp.float32),
            jax.ShapeDtypeStruct((n, D), jnp.float32),
        ],
    )(x, wa, wb)


def _sc_gather(a, b, src, dst):
    e = src.shape[0]
    info = plsc.get_sparse_core_info()
    nc, ns = info.num_cores, info.num_subcores
    nw = nc * ns
    total_chunks = e // CH
    iters = (total_chunks + nw - 1) // nw
    mesh = plsc.VectorSubcoreMesh(core_axis_name="c", subcore_axis_name="s")

    @functools.partial(
        pl.kernel,
        mesh=mesh,
        out_type=jax.ShapeDtypeStruct((e, D), jnp.float32),
        scratch_types=[
            pltpu.VMEM((2, CH), jnp.int32),
            pltpu.VMEM((2, CH), jnp.int32),
            pltpu.VMEM((2, CH, D), jnp.float32),
            pltpu.VMEM((2, CH, D), jnp.float32),
        ] + [pltpu.SemaphoreType.DMA] * 6,
    )
    def gather_kernel(a_hbm, b_hbm, src_hbm, dst_hbm, g_hbm,
                      idxs, idxd, bufa, bufb,
                      sga0, sga1, sgb0, sgb1, ssa0, ssa1):
        sga = (sga0, sga1)
        sgb = (sgb0, sgb1)
        ssa = (ssa0, ssa1)
        wid = lax.axis_index("s") * nc + lax.axis_index("c")

        def valid(i):
            return wid + i * nw < total_chunks

        def stage_and_gather(i, slot):
            base = (wid + i * nw) * CH
            pltpu.sync_copy(src_hbm.at[pl.ds(base, CH)], idxs.at[slot])
            pltpu.sync_copy(dst_hbm.at[pl.ds(base, CH)], idxd.at[slot])
            pltpu.async_copy(a_hbm.at[idxs.at[slot]], bufa.at[slot], sga[slot])
            pltpu.async_copy(b_hbm.at[idxd.at[slot]], bufb.at[slot], sgb[slot])

        def wait_gathers(slot):
            pltpu.make_async_copy(a_hbm.at[pl.ds(0, CH)], bufa.at[slot], sga[slot]).wait()
            pltpu.make_async_copy(b_hbm.at[pl.ds(0, CH)], bufb.at[slot], sgb[slot]).wait()

        def add_rows(slot):
            def row(r, carry):
                for c in range(D // 16):
                    sl = pl.ds(c * 16, 16)
                    bufa[slot, r, sl] = bufa[slot, r, sl] + bufb[slot, r, sl]
                return carry

            lax.fori_loop(0, CH, row, 0)

        def scatter(i, slot):
            base = (wid + i * nw) * CH
            pltpu.async_copy(bufa.at[slot], g_hbm.at[pl.ds(base, CH)], ssa[slot])

        def wait_scatters(slot):
            pltpu.make_async_copy(bufa.at[slot], g_hbm.at[pl.ds(0, CH)], ssa[slot]).wait()

        def step(i, slot, first):
            nslot = 1 - slot

            @pl.when(valid(i))
            def _():
                wait_gathers(slot)

            @pl.when(valid(i + 1))
            def _():
                if not first:
                    wait_scatters(nslot)
                stage_and_gather(i + 1, nslot)

            @pl.when(valid(i))
            def _():
                add_rows(slot)
                scatter(i, slot)

        stage_and_gather(0, 0)
        step(0, 0, first=True)

        def loop_body(j, carry):
            step(2 * j + 1, 1, first=False)
            step(2 * j + 2, 0, first=False)
            return carry

        lax.fori_loop(0, (iters + 1) // 2, loop_body, 0)

        wait_scatters(0)
        wait_scatters(1)

    return gather_kernel(a, b, src, dst)


def _mlp_body(prev_ref, g_ref, e_ref, wc_ref, b1_ref, g1_ref, be1_ref,
              w2_ref, b2_ref, o_ref):
    del prev_ref
    eb = e_ref[...]
    h = (g_ref[...]
         + jnp.dot(eb, wc_ref[...], preferred_element_type=jnp.float32)
         + b1_ref[...])
    m = jnp.mean(h, axis=-1, keepdims=True)
    v = jnp.mean((h - m) ** 2, axis=-1, keepdims=True)
    hn = (h - m) / jnp.sqrt(v + EPS) * g1_ref[...] + be1_ref[...]
    hr = jnp.maximum(hn, 0.0)
    o_ref[...] = (eb + jnp.dot(hr, w2_ref[...], preferred_element_type=jnp.float32)
                  + b2_ref[...])


def _edge_mlp_chunk(out_prev, g, edge, wc, b1, g1, be1, w2, b2, blk0, nblk):
    e = edge.shape[0]
    row = lambda v: v.reshape(1, D)
    vec_spec = pl.BlockSpec((1, D), lambda i: (0, 0))
    mat_spec = pl.BlockSpec((D, D), lambda i: (0, 0))
    out_spec = pl.BlockSpec((BE, D), lambda i: (blk0 + i, 0))
    body = _mlp_body if out_prev is not None else functools.partial(_mlp_body, None)
    prev_spec = ([pl.BlockSpec(memory_space=pl.ANY)]
                 if out_prev is not None else [])
    prev_arg = (out_prev,) if out_prev is not None else ()
    return pl.pallas_call(
        body,
        grid=(nblk,),
        in_specs=prev_spec + [
            pl.BlockSpec((BE, D), lambda i: (i, 0)),
            out_spec, mat_spec,
            vec_spec, vec_spec, vec_spec, mat_spec, vec_spec],
        out_specs=out_spec,
        out_shape=jax.ShapeDtypeStruct((e, D), jnp.float32),
        input_output_aliases={0: 0} if out_prev is not None else {},
    )(*prev_arg, g, edge, wc, row(b1), row(g1), row(be1), w2, row(b2))


def kernel(x, edge_index, edge, W1, b1, g1, be1, W2, b2):
    e = edge.shape[0]
    src = edge_index[0]
    dst = edge_index[1]
    wa, wb, wc = W1[:D], W1[D:2 * D], W1[2 * D:]
    a, b = _node_projections(x, wa, wb)

    sizes = CHUNK_SIZES if sum(CHUNK_SIZES) == e else (e,)
    assert all(ec % BE == 0 and ec % CH == 0 for ec in sizes)
    out = None
    e0 = 0
    for ec in sizes:
        sl = slice(e0, e0 + ec)
        g = _sc_gather(a, b, src[sl], dst[sl])
        out = _edge_mlp_chunk(out, g, edge, wc, b1, g1, be1, W2, b2,
                              e0 // BE, ec // BE)
        e0 += ec
    return out


# speedup vs baseline: 1.5278x; 1.0448x over previous
import functools

import jax
import jax.numpy as jnp
from jax import lax
from jax.experimental import pallas as pl
from jax.experimental.pallas import tpu as pltpu
from jax.experimental.pallas import tpu_sc as plsc

EPS = 1e-5
D = 128
CH = 128
CHUNK_SIZES = (32000, 96000, 96000, 64000, 32000)
BE = 4000


def _proj_body(x_ref, wa_ref, wb_ref, a_ref, b_ref):
    x = x_ref[...]
    a_ref[...] = jnp.dot(x, wa_ref[...], preferred_element_type=jnp.float32)
    b_ref[...] = jnp.dot(x, wb_ref[...], preferred_element_type=jnp.float32)


def _node_projections(x, wa, wb):
    n = x.shape[0]
    bn = 2000 if n % 2000 == 0 else n
    grid = n // bn
    return pl.pallas_call(
        _proj_body,
        grid=(grid,),
        in_specs=[
            pl.BlockSpec((bn, D), lambda i: (i, 0)),
            pl.BlockSpec((D, D), lambda i: (0, 0)),
            pl.BlockSpec((D, D), lambda i: (0, 0)),
        ],
        out_specs=[
            pl.BlockSpec((bn, D), lambda i: (i, 0)),
            pl.BlockSpec((bn, D), lambda i: (i, 0)),
        ],
        out_shape=[
            jax.ShapeDtypeStruct((n, D), jnp.float32),
            jax.ShapeDtypeStruct((n, D), jnp.float32),
        ],
    )(x, wa, wb)


def _sc_gather(a, b, eii):
    e = eii.shape[0] * CH
    info = plsc.get_sparse_core_info()
    nc, ns = info.num_cores, info.num_subcores
    nw = nc * ns
    total_chunks = e // CH
    iters = (total_chunks + nw - 1) // nw
    mesh = plsc.VectorSubcoreMesh(core_axis_name="c", subcore_axis_name="s")

    @functools.partial(
        pl.kernel,
        mesh=mesh,
        out_type=jax.ShapeDtypeStruct((e, D), jnp.float32),
        scratch_types=[
            pltpu.VMEM((2, 2, CH), jnp.int32),
            pltpu.VMEM((2, CH, D), jnp.float32),
            pltpu.VMEM((2, CH, D), jnp.float32),
        ] + [pltpu.SemaphoreType.DMA] * 6,
    )
    def gather_kernel(a_hbm, b_hbm, ei_hbm, g_hbm,
                      idx2, bufa, bufb,
                      sga0, sga1, sgb0, sgb1, ssa0, ssa1):
        sga = (sga0, sga1)
        sgb = (sgb0, sgb1)
        ssa = (ssa0, ssa1)
        wid = lax.axis_index("s") * nc + lax.axis_index("c")

        def valid(i):
            return wid + i * nw < total_chunks

        def stage_and_gather(i, slot):
            k = wid + i * nw
            base = k * CH
            pltpu.sync_copy(ei_hbm.at[k], idx2.at[slot])
            pltpu.async_copy(a_hbm.at[idx2.at[slot, 0]], bufa.at[slot], sga[slot])
            pltpu.async_copy(b_hbm.at[idx2.at[slot, 1]], bufb.at[slot], sgb[slot])

        def wait_gathers(slot):
            pltpu.make_async_copy(a_hbm.at[pl.ds(0, CH)], bufa.at[slot], sga[slot]).wait()
            pltpu.make_async_copy(b_hbm.at[pl.ds(0, CH)], bufb.at[slot], sgb[slot]).wait()

        def add_rows(slot):
            def row(r, carry):
                for c in range(D // 16):
                    sl = pl.ds(c * 16, 16)
                    bufa[slot, r, sl] = bufa[slot, r, sl] + bufb[slot, r, sl]
                return carry

            lax.fori_loop(0, CH, row, 0)

        def scatter(i, slot):
            base = (wid + i * nw) * CH
            pltpu.async_copy(bufa.at[slot], g_hbm.at[pl.ds(base, CH)], ssa[slot])

        def wait_scatters(slot):
            pltpu.make_async_copy(bufa.at[slot], g_hbm.at[pl.ds(0, CH)], ssa[slot]).wait()

        def step(i, slot, first):
            nslot = 1 - slot

            @pl.when(valid(i))
            def _():
                wait_gathers(slot)

            @pl.when(valid(i + 1))
            def _():
                if not first:
                    wait_scatters(nslot)
                stage_and_gather(i + 1, nslot)

            @pl.when(valid(i))
            def _():
                add_rows(slot)
                scatter(i, slot)

        stage_and_gather(0, 0)
        step(0, 0, first=True)

        def loop_body(j, carry):
            step(2 * j + 1, 1, first=False)
            step(2 * j + 2, 0, first=False)
            return carry

        lax.fori_loop(0, (iters + 1) // 2, loop_body, 0)

        wait_scatters(0)
        wait_scatters(1)

    return gather_kernel(a, b, eii)


def _mlp_body(prev_ref, g_ref, e_ref, wc_ref, b1_ref, g1_ref, be1_ref,
              w2_ref, b2_ref, o_ref):
    del prev_ref
    eb = e_ref[...]
    h = (g_ref[...]
         + jnp.dot(eb, wc_ref[...], preferred_element_type=jnp.float32)
         + b1_ref[...])
    m = jnp.mean(h, axis=-1, keepdims=True)
    v = jnp.mean((h - m) ** 2, axis=-1, keepdims=True)
    hn = (h - m) / jnp.sqrt(v + EPS) * g1_ref[...] + be1_ref[...]
    hr = jnp.maximum(hn, 0.0)
    o_ref[...] = (eb + jnp.dot(hr, w2_ref[...], preferred_element_type=jnp.float32)
                  + b2_ref[...])


def _edge_mlp_chunk(out_prev, g, edge, wc, b1, g1, be1, w2, b2, blk0, nblk):
    e = edge.shape[0]
    row = lambda v: v.reshape(1, D)
    vec_spec = pl.BlockSpec((1, D), lambda i: (0, 0))
    mat_spec = pl.BlockSpec((D, D), lambda i: (0, 0))
    out_spec = pl.BlockSpec((BE, D), lambda i: (blk0 + i, 0))
    body = _mlp_body if out_prev is not None else functools.partial(_mlp_body, None)
    prev_spec = ([pl.BlockSpec(memory_space=pl.ANY)]
                 if out_prev is not None else [])
    prev_arg = (out_prev,) if out_prev is not None else ()
    return pl.pallas_call(
        body,
        grid=(nblk,),
        in_specs=prev_spec + [
            pl.BlockSpec((BE, D), lambda i: (i, 0)),
            out_spec, mat_spec,
            vec_spec, vec_spec, vec_spec, mat_spec, vec_spec],
        out_specs=out_spec,
        out_shape=jax.ShapeDtypeStruct((e, D), jnp.float32),
        input_output_aliases={0: 0} if out_prev is not None else {},
    )(*prev_arg, g, edge, wc, row(b1), row(g1), row(be1), w2, row(b2))


def kernel(x, edge_index, edge, W1, b1, g1, be1, W2, b2):
    e = edge.shape[0]
    wa, wb, wc = W1[:D], W1[D:2 * D], W1[2 * D:]
    a, b = _node_projections(x, wa, wb)
    eii = jnp.stack([edge_index[0].reshape(-1, CH),
                     edge_index[1].reshape(-1, CH)], axis=1)

    sizes = CHUNK_SIZES if sum(CHUNK_SIZES) == e else (e,)
    assert all(ec % BE == 0 and ec % CH == 0 for ec in sizes)
    out = None
    e0 = 0
    for ec in sizes:
        g = _sc_gather(a, b, eii[e0 // CH:(e0 + ec) // CH])
        out = _edge_mlp_chunk(out, g, edge, wc, b1, g1, be1, W2, b2,
                              e0 // BE, ec // BE)
        e0 += ec
    return out


# speedup vs baseline: 1.5460x; 1.0119x over previous
import functools

import jax
import jax.numpy as jnp
from jax import lax
from jax.experimental import pallas as pl
from jax.experimental.pallas import tpu as pltpu
from jax.experimental.pallas import tpu_sc as plsc

EPS = 1e-5
D = 128
CH = 128
CHUNK_SIZES = (16000, 80000, 64000, 64000, 48000, 48000)
BE = 8000


def _proj_body(x_ref, wa_ref, wb_ref, a_ref, b_ref):
    x = x_ref[...]
    a_ref[...] = jnp.dot(x, wa_ref[...], preferred_element_type=jnp.float32)
    b_ref[...] = jnp.dot(x, wb_ref[...], preferred_element_type=jnp.float32)


def _node_projections(x, wa, wb):
    n = x.shape[0]
    bn = 2000 if n % 2000 == 0 else n
    grid = n // bn
    return pl.pallas_call(
        _proj_body,
        grid=(grid,),
        in_specs=[
            pl.BlockSpec((bn, D), lambda i: (i, 0)),
            pl.BlockSpec((D, D), lambda i: (0, 0)),
            pl.BlockSpec((D, D), lambda i: (0, 0)),
        ],
        out_specs=[
            pl.BlockSpec((bn, D), lambda i: (i, 0)),
            pl.BlockSpec((bn, D), lambda i: (i, 0)),
        ],
        out_shape=[
            jax.ShapeDtypeStruct((n, D), jnp.float32),
            jax.ShapeDtypeStruct((n, D), jnp.float32),
        ],
    )(x, wa, wb)


def _sc_gather(a, b, eii):
    e = eii.shape[0] * CH
    info = plsc.get_sparse_core_info()
    nc, ns = info.num_cores, info.num_subcores
    nw = nc * ns
    total_chunks = e // CH
    iters = (total_chunks + nw - 1) // nw
    mesh = plsc.VectorSubcoreMesh(core_axis_name="c", subcore_axis_name="s")

    @functools.partial(
        pl.kernel,
        mesh=mesh,
        out_type=jax.ShapeDtypeStruct((e, D), jnp.float32),
        scratch_types=[
            pltpu.VMEM((2, 2, CH), jnp.int32),
            pltpu.VMEM((2, CH, D), jnp.float32),
            pltpu.VMEM((2, CH, D), jnp.float32),
        ] + [pltpu.SemaphoreType.DMA] * 6,
    )
    def gather_kernel(a_hbm, b_hbm, ei_hbm, g_hbm,
                      idx2, bufa, bufb,
                      sga0, sga1, sgb0, sgb1, ssa0, ssa1):
        sga = (sga0, sga1)
        sgb = (sgb0, sgb1)
        ssa = (ssa0, ssa1)
        wid = lax.axis_index("s") * nc + lax.axis_index("c")

        def valid(i):
            return wid + i * nw < total_chunks

        def stage_and_gather(i, slot):
            k = wid + i * nw
            base = k * CH
            pltpu.sync_copy(ei_hbm.at[k], idx2.at[slot])
            pltpu.async_copy(a_hbm.at[idx2.at[slot, 0]], bufa.at[slot], sga[slot])
            pltpu.async_copy(b_hbm.at[idx2.at[slot, 1]], bufb.at[slot], sgb[slot])

        def wait_gathers(slot):
            pltpu.make_async_copy(a_hbm.at[pl.ds(0, CH)], bufa.at[slot], sga[slot]).wait()
            pltpu.make_async_copy(b_hbm.at[pl.ds(0, CH)], bufb.at[slot], sgb[slot]).wait()

        def add_rows(slot):
            def row(r, carry):
                for c in range(D // 16):
                    sl = pl.ds(c * 16, 16)
                    bufa[slot, r, sl] = bufa[slot, r, sl] + bufb[slot, r, sl]
                return carry

            lax.fori_loop(0, CH, row, 0)

        def scatter(i, slot):
            base = (wid + i * nw) * CH
            pltpu.async_copy(bufa.at[slot], g_hbm.at[pl.ds(base, CH)], ssa[slot])

        def wait_scatters(slot):
            pltpu.make_async_copy(bufa.at[slot], g_hbm.at[pl.ds(0, CH)], ssa[slot]).wait()

        def step(i, slot, first):
            nslot = 1 - slot

            @pl.when(valid(i))
            def _():
                wait_gathers(slot)

            @pl.when(valid(i + 1))
            def _():
                if not first:
                    wait_scatters(nslot)
                stage_and_gather(i + 1, nslot)

            @pl.when(valid(i))
            def _():
                add_rows(slot)
                scatter(i, slot)

        stage_and_gather(0, 0)
        step(0, 0, first=True)

        def loop_body(j, carry):
            step(2 * j + 1, 1, first=False)
            step(2 * j + 2, 0, first=False)
            return carry

        lax.fori_loop(0, (iters + 1) // 2, loop_body, 0)

        wait_scatters(0)
        wait_scatters(1)

    return gather_kernel(a, b, eii)


def _mlp_body(prev_ref, g_ref, e_ref, wc_ref, b1_ref, g1_ref, be1_ref,
              w2_ref, b2_ref, o_ref):
    del prev_ref
    eb = e_ref[...]
    h = (g_ref[...]
         + jnp.dot(eb, wc_ref[...], preferred_element_type=jnp.float32)
         + b1_ref[...])
    m = jnp.mean(h, axis=-1, keepdims=True)
    v = jnp.mean((h - m) ** 2, axis=-1, keepdims=True)
    rs = jax.lax.rsqrt(v + EPS)
    hn = (h - m) * (rs * g1_ref[...]) + be1_ref[...]
    hr = jnp.maximum(hn, 0.0)
    o_ref[...] = (eb + jnp.dot(hr, w2_ref[...], preferred_element_type=jnp.float32)
                  + b2_ref[...])


def _edge_mlp_chunk(out_prev, g, edge, wc, b1, g1, be1, w2, b2, blk0, nblk):
    e = edge.shape[0]
    row = lambda v: v.reshape(1, D)
    vec_spec = pl.BlockSpec((1, D), lambda i: (0, 0))
    mat_spec = pl.BlockSpec((D, D), lambda i: (0, 0))
    out_spec = pl.BlockSpec((BE, D), lambda i: (blk0 + i, 0))
    body = _mlp_body if out_prev is not None else functools.partial(_mlp_body, None)
    prev_spec = ([pl.BlockSpec(memory_space=pl.ANY)]
                 if out_prev is not None else [])
    prev_arg = (out_prev,) if out_prev is not None else ()
    return pl.pallas_call(
        body,
        grid=(nblk,),
        in_specs=prev_spec + [
            pl.BlockSpec((BE, D), lambda i: (i, 0)),
            out_spec, mat_spec,
            vec_spec, vec_spec, vec_spec, mat_spec, vec_spec],
        out_specs=out_spec,
        out_shape=jax.ShapeDtypeStruct((e, D), jnp.float32),
        input_output_aliases={0: 0} if out_prev is not None else {},
    )(*prev_arg, g, edge, wc, row(b1), row(g1), row(be1), w2, row(b2))


def kernel(x, edge_index, edge, W1, b1, g1, be1, W2, b2):
    e = edge.shape[0]
    wa, wb, wc = W1[:D], W1[D:2 * D], W1[2 * D:]
    a, b = _node_projections(x, wa, wb)
    eii = jnp.stack([edge_index[0].reshape(-1, CH),
                     edge_index[1].reshape(-1, CH)], axis=1)

    sizes = CHUNK_SIZES if sum(CHUNK_SIZES) == e else (e,)
    assert all(ec % BE == 0 and ec % CH == 0 for ec in sizes)
    out = None
    e0 = 0
    for ec in sizes:
        g = _sc_gather(a, b, eii[e0 // CH:(e0 + ec) // CH])
        out = _edge_mlp_chunk(out, g, edge, wc, b1, g1, be1, W2, b2,
                              e0 // BE, ec // BE)
        e0 += ec
    return out
